# baseline (speedup 1.0000x reference)
import functools

import jax
import jax.numpy as jnp
from jax import lax
from jax.experimental import pallas as pl
from jax.experimental.pallas import tpu as pltpu
from jax.experimental.pallas import tpu_sc as plsc

B = 16384
D = 64
LANES = 16
V = 1_000_000
TCS = (V + 127) // 128
CHT = 4
CW = CHT * 128
MAX_OFF = (TCS * 128) - CW
LCAP = 1040
CCAP = 96
PEND = 128
PCAP = PEND + LANES
FLUSH_AT = PEND - CCAP
NPC = 2048
NPIECE = B // NPC
NROW = B + 16


def _make_sc_gather(nc: int, ns: int):
    nw = nc * ns
    span_max = (TCS + nw - 1) // nw
    n_chunks = (span_max + CHT - 1) // CHT

    mesh = plsc.VectorSubcoreMesh(core_axis_name="c", subcore_axis_name="s")

    @functools.partial(
        pl.kernel,
        mesh=mesh,
        compiler_params=pltpu.CompilerParams(
            use_tc_tiling_on_sc=True, needs_layout_passes=False),
        out_type=(
            jax.ShapeDtypeStruct((NROW, 128), jnp.float32),
            jax.ShapeDtypeStruct((NROW, 128), jnp.float32),
        ),
        scratch_types=[
            pltpu.VMEM((NPC,), jnp.int32),
            pltpu.VMEM((LCAP,), jnp.int32),
            pltpu.VMEM((LCAP,), jnp.int32),
            pltpu.VMEM((LCAP,), jnp.int32),
            pltpu.VMEM((LCAP,), jnp.int32),
            pltpu.VMEM((CCAP,), jnp.int32),
            pltpu.VMEM((CCAP,), jnp.int32),
            pltpu.VMEM((D, CW), jnp.float32),
            pltpu.VMEM((D, CW), jnp.float32),
            pltpu.VMEM((PEND, 128), jnp.float32),
            pltpu.VMEM((PEND, 128), jnp.float32),
            pltpu.VMEM((PCAP,), jnp.int32),
            pltpu.VMEM((PCAP,), jnp.int32),
            pltpu.VMEM((PEND,), jnp.int32),
            pltpu.SemaphoreType.DMA,
            pltpu.SemaphoreType.DMA,
        ],
    )
    def sc_gather(wit_hbm, wct_hbm, iw_hbm, cw_hbm, ga_hbm, gb_hbm,
                  ibuf, la_c, la_p, lb_c, lb_p, cc_c, cc_p,
                  buf_a, buf_b, pr_a, pr_b, pp_a, pp_b, spos,
                  sem_in, sem_out):
        wid = lax.axis_index("s") * nc + lax.axis_index("c")
        base_tc = (wid * TCS) // nw
        end_tc = ((wid + 1) * TCS) // nw
        lo = base_tc * 128
        hi = jnp.minimum(end_tc * 128, V)

        lane = lax.iota(jnp.int32, LANES)

        def scal(vec, l):
            return lax.reduce_sum_p.bind(
                jnp.where(lane == l, vec, 0), axes=(0,))

        def scan(src_hbm, dst_c, dst_p):
            def piece(pi, cnt0):
                pltpu.sync_copy(src_hbm.at[pl.ds(pi * NPC, NPC)], ibuf)

                def body(q, cnt):
                    v = ibuf[pl.ds(q * LANES, LANES)]
                    m = (v >= lo) & (v < hi)
                    pc_vec = plsc.all_reduce_population_count(m)
                    pc = lax.reduce_max_p.bind(pc_vec, axes=(0,))
                    plsc.store_compressed(
                        dst_c.at[pl.ds(cnt, LANES)], v, mask=m)
                    plsc.store_compressed(
                        dst_p.at[pl.ds(cnt, LANES)],
                        pi * NPC + q * LANES + lane, mask=m)
                    return cnt + pc
                return lax.fori_loop(0, NPC // LANES, body, cnt0)
            return lax.fori_loop(0, NPIECE, piece, jnp.int32(0))

        cnt_a = scan(iw_hbm, la_c, la_p)
        cnt_b = scan(cw_hbm, lb_c, lb_p)

        def subfilter(src_c, src_p, cnt, clo, chi):
            nv = (cnt + LANES - 1) // LANES

            def body(q, k):
                cv = src_c[pl.ds(q * LANES, LANES)]
                pv = src_p[pl.ds(q * LANES, LANES)]
                valid = (q * LANES + lane) < cnt
                m = (cv >= clo) & (cv < chi) & valid
                pc_vec = plsc.all_reduce_population_count(m)
                pc = lax.reduce_max_p.bind(pc_vec, axes=(0,))
                plsc.store_compressed(cc_c.at[pl.ds(k, LANES)], cv, mask=m)
                plsc.store_compressed(cc_p.at[pl.ds(k, LANES)], pv, mask=m)
                return k + pc
            return lax.fori_loop(0, nv, body, jnp.int32(0))

        def extract_append(buf, pr, pp, off, pend, n):
            def body(e, k):
                qv = (e // LANES) * LANES
                l = e % LANES
                col = scal(cc_c[pl.ds(qv, LANES)], l)
                c_off = jnp.full((LANES,), 0, jnp.int32) + (col - off)
                slot = pend + e
                for c in range(D // LANES):
                    jv = c * LANES + lane
                    v = plsc.load_gather(buf, [jv, c_off])
                    pr[slot, pl.ds(c * LANES, LANES)] = v
                return k
            lax.fori_loop(0, n, body, jnp.int32(0))

            def cpos(q, k):
                pp[pl.ds(pend + q * LANES, LANES)] = \
                    cc_p[pl.ds(q * LANES, LANES)]
                return k
            lax.fori_loop(0, (n + LANES - 1) // LANES, cpos, jnp.int32(0))
            return pend + n

        def flush(pr, pp, g_hbm, pend):
            def trash(t, k):
                pp[pl.ds(pend + t * LANES, LANES)] = \
                    jnp.full((LANES,), B, jnp.int32)
                return k
            nt = (PEND - pend + LANES - 1) // LANES
            lax.fori_loop(0, nt, trash, jnp.int32(0))

            def cp(q, k):
                spos[pl.ds(q * LANES, LANES)] = pp[pl.ds(q * LANES, LANES)]
                return k
            lax.fori_loop(0, PEND // LANES, cp, jnp.int32(0))
            pltpu.async_copy(pr, g_hbm.at[spos], sem_out).wait()

        def chunk(c, carry):
            pend_a, pend_b = carry
            tc0 = base_tc + c * CHT
            clo = tc0 * 128
            chi = jnp.minimum(clo + CW, hi)
            off = pl.multiple_of(jnp.minimum(clo, MAX_OFF), 128)
            copies = []
            for k in range(D // 8):
                copies.append(pltpu.async_copy(
                    wit_hbm.at[pl.ds(8 * k, 8), pl.ds(off, CW)],
                    buf_a.at[pl.ds(8 * k, 8), :], sem_in))
                copies.append(pltpu.async_copy(
                    wct_hbm.at[pl.ds(8 * k, 8), pl.ds(off, CW)],
                    buf_b.at[pl.ds(8 * k, 8), :], sem_in))
            for cp_ in copies:
                cp_.wait()

            na = subfilter(la_c, la_p, cnt_a, clo, chi)
            pend_a = extract_append(buf_a, pr_a, pp_a, off, pend_a, na)
            nb = subfilter(lb_c, lb_p, cnt_b, clo, chi)
            pend_b = extract_append(buf_b, pr_b, pp_b, off, pend_b, nb)

            @pl.when(pend_a > FLUSH_AT)
            def _():
                flush(pr_a, pp_a, ga_hbm, pend_a)

            @pl.when(pend_b > FLUSH_AT)
            def _():
                flush(pr_b, pp_b, gb_hbm, pend_b)

            pend_a = jnp.where(pend_a > FLUSH_AT, 0, pend_a)
            pend_b = jnp.where(pend_b > FLUSH_AT, 0, pend_b)
            return pend_a, pend_b

        pend_a, pend_b = lax.fori_loop(
            0, n_chunks, chunk, (jnp.int32(0), jnp.int32(0)))

        @pl.when(pend_a > 0)
        def _():
            flush(pr_a, pp_a, ga_hbm, pend_a)

        @pl.when(pend_b > 0)
        def _():
            flush(pr_b, pp_b, gb_hbm, pend_b)

    return sc_gather


def _loss_body(a_ref, b_ref, o_ref):
    prod = a_ref[...] * b_ref[...]
    s = jnp.sum(prod[:, :D], axis=1)
    ls = jnp.minimum(s, 0.0) - jnp.log1p(jnp.exp(-jnp.abs(s)))
    o_ref[0, 0] = -jnp.sum(ls) * (1.0 / B)


def kernel(W_input, W_context, input_word, context_word):
    info = plsc.get_sparse_core_info()
    nc, ns = info.num_cores, info.num_subcores

    iw = input_word.astype(jnp.int32)
    cw = context_word.astype(jnp.int32)

    ga, gb = _make_sc_gather(nc, ns)(W_input.T, W_context.T, iw, cw)

    loss = pl.pallas_call(
        _loss_body,
        out_shape=jax.ShapeDtypeStruct((1, 1), jnp.float32),
        out_specs=pl.BlockSpec(memory_space=pltpu.SMEM),
    )(ga[:B], gb[:B])
    return loss.reshape(())


# speedup vs baseline: 3.2408x; 3.2408x over previous
import functools

import jax
import jax.numpy as jnp
from jax import lax
from jax.experimental import pallas as pl
from jax.experimental.pallas import tpu as pltpu
from jax.experimental.pallas import tpu_sc as plsc

B = 16384
D = 64
LANES = 16
BP = 256


def _make_sc_dots(nc: int, ns: int):
    nw = nc * ns
    b_per_w = B // nw
    n_passes = b_per_w // BP
    gpp = BP // LANES

    mesh = plsc.VectorSubcoreMesh(core_axis_name="c", subcore_axis_name="s")

    @functools.partial(
        pl.kernel,
        mesh=mesh,
        compiler_params=pltpu.CompilerParams(
            use_tc_tiling_on_sc=True, needs_layout_passes=False),
        out_type=jax.ShapeDtypeStruct((B,), jnp.float32),
        scratch_types=[
            pltpu.VMEM((b_per_w,), jnp.int32),
            pltpu.VMEM((b_per_w,), jnp.int32),
            pltpu.VMEM((2 * LANES, D), jnp.float32),
            pltpu.VMEM((2 * LANES, D), jnp.float32),
            pltpu.VMEM((BP, 128), jnp.float32),
            pltpu.VMEM((BP, 128), jnp.float32),
            pltpu.VMEM((b_per_w,), jnp.float32),
            pltpu.SemaphoreType.DMA,
            pltpu.SemaphoreType.DMA,
        ],
    )
    def sc_dots(wi_hbm, wc_hbm, iw_hbm, cw_hbm, dummy_hbm, out_hbm,
                idx_a, idx_b, stage_a, stage_b, rows_a, rows_b, dots,
                sem0, sem1):
        wid = lax.axis_index("s") * nc + lax.axis_index("c")
        base = wid * b_per_w
        pltpu.sync_copy(iw_hbm.at[pl.ds(base, b_per_w)], idx_a)
        pltpu.sync_copy(cw_hbm.at[pl.ds(base, b_per_w)], idx_b)

        lane = lax.iota(jnp.int32, LANES)

        def fire(pb, u, pp, sem):
            va = idx_a[pl.ds(pb + u * LANES, LANES)]
            vb = idx_b[pl.ds(pb + u * LANES, LANES)]
            soff = pp * LANES
            for l in range(LANES):
                ra = lax.reduce_sum_p.bind(
                    jnp.where(lane == l, va, 0), axes=(0,))
                rb = lax.reduce_sum_p.bind(
                    jnp.where(lane == l, vb, 0), axes=(0,))
                pltpu.async_copy(
                    wi_hbm.at[pl.ds(ra, 1), :],
                    stage_a.at[pl.ds(soff + l, 1), :], sem)
                pltpu.async_copy(
                    wc_hbm.at[pl.ds(rb, 1), :],
                    stage_b.at[pl.ds(soff + l, 1), :], sem)

        def drain_and_copy(u, pp, sem):
            soff = pp * LANES
            pltpu.make_async_copy(
                dummy_hbm, stage_a.at[pl.ds(soff, LANES), :], sem).wait()
            pltpu.make_async_copy(
                dummy_hbm, stage_b.at[pl.ds(soff, LANES), :], sem).wait()
            for l in range(LANES):
                dst_a = rows_a.at[u * LANES + l]
                dst_b = rows_b.at[u * LANES + l]
                for c in range(D // LANES):
                    sl = pl.ds(c * LANES, LANES)
                    dst_a[sl] = stage_a[soff + l, sl]
                    dst_b[sl] = stage_b[soff + l, sl]

        def run_pass(p, carry):
            pb = p * BP

            def pipe(u, c):
                @pl.when(u < gpp)
                def _():
                    @pl.when(u % 2 == 0)
                    def _():
                        fire(pb, u, 0, sem0)

                    @pl.when(u % 2 == 1)
                    def _():
                        fire(pb, u, 1, sem1)

                @pl.when(u > 0)
                def _():
                    @pl.when((u - 1) % 2 == 0)
                    def _():
                        drain_and_copy(u - 1, 0, sem0)

                    @pl.when((u - 1) % 2 == 1)
                    def _():
                        drain_and_copy(u - 1, 1, sem1)

                return c

            lax.fori_loop(0, gpp + 1, pipe, 0)

            def compute(g, c):
                kv = g * LANES + lane
                acc = jnp.zeros((LANES,), jnp.float32)
                for j in range(D):
                    jv = jnp.full((LANES,), j, jnp.int32)
                    va = plsc.load_gather(rows_a, [kv, jv])
                    vb = plsc.load_gather(rows_b, [kv, jv])
                    acc = acc + va * vb
                dots[pl.ds(pb + g * LANES, LANES)] = acc
                return c

            lax.fori_loop(0, gpp, compute, 0)
            return carry

        lax.fori_loop(0, n_passes, run_pass, 0)
        pltpu.sync_copy(dots, out_hbm.at[pl.ds(base, b_per_w)])

    return sc_dots


def _loss_body(x_ref, o_ref):
    x = x_ref[...]
    ls = jnp.minimum(x, 0.0) - jnp.log1p(jnp.exp(-jnp.abs(x)))
    o_ref[0, 0] = -jnp.sum(ls) * (1.0 / B)


def kernel(W_input, W_context, input_word, context_word):
    info = plsc.get_sparse_core_info()
    nc, ns = info.num_cores, info.num_subcores

    iw = input_word.astype(jnp.int32)
    cw = context_word.astype(jnp.int32)
    dummy = jnp.zeros((LANES, D), jnp.float32)

    dots = _make_sc_dots(nc, ns)(W_input, W_context, iw, cw, dummy)

    loss = pl.pallas_call(
        _loss_body,
        out_shape=jax.ShapeDtypeStruct((1, 1), jnp.float32),
        out_specs=pl.BlockSpec(memory_space=pltpu.SMEM),
    )(dots.reshape(B // 128, 128))
    return loss.reshape(())
